# Initial kernel scaffold; baseline (speedup 1.0000x reference)
import jax
import jax.numpy as jnp
from jax.experimental import pallas as pl


def kernel(u, v, time_diff, event_types, t_bar, t, z0, omega_w, omega_b, w_t, alpha, psi, W_struct_w, W_struct_b, W_rec_w, W_rec_b, W_t_w, W_t_b):
    raise NotImplementedError("write your pallas kernel here")


# speedup vs baseline: 67.0024x; 67.0024x over previous
import numpy as np
import jax
import jax.numpy as jnp
from jax.experimental import pallas as pl
from jax.experimental.pallas import tpu as pltpu

N = 10000
H = 256
B = 200
TD_MAX = 100.0


def _scan_kernel(u_ref, v_ref, et_ref, t_ref, wt_ref, al_ref, ps_ref, ob_ref,
                 td_ref, tb_ref, z0_ref, wa_ref, wb_ref, WsT_ref, WrT_ref,
                 WtT_ref, bias_ref, lam_ref, z_ref, s_ref):
    i = pl.program_id(0)

    @pl.when(i == 0)
    def _init():
        z_ref[...] = z0_ref[...]
        s_ref[...] = jax.lax.dot_general(
            wb_ref[...], z0_ref[...],
            dimension_numbers=(((1,), (1,)), ((), ())),
            preferred_element_type=jnp.float32)

    ui = u_ref[i]
    vi = v_ref[i]
    et = et_ref[i]
    ti = t_ref[i]
    wtv = wt_ref[et]
    alv = al_ref[et]
    psv = ps_ref[et]
    bv = ob_ref[et]

    zu = z_ref[pl.ds(ui, 1), :]
    zv = z_ref[pl.ds(vi, 1), :]

    tdn = td_ref[pl.ds(i, 1), :]
    td01 = jnp.concatenate([tdn[:, 0:4], tdn[:, 4:8]], axis=0)
    m_struct = jnp.concatenate([zv, zu], axis=0)
    m_rec = jnp.concatenate([zu, zv], axis=0)
    h2 = jax.nn.sigmoid(
        jnp.dot(m_struct, WsT_ref[...], preferred_element_type=jnp.float32)
        + jnp.dot(m_rec, WrT_ref[...], preferred_element_type=jnp.float32)
        + jnp.dot(td01, WtT_ref[...], preferred_element_type=jnp.float32)
        + bias_ref[...])

    wa_et = wa_ref[pl.ds(et, 1), :]
    a_u = jnp.sum(zu * wa_et)
    a_v = jnp.sum(zv * wa_et)
    s_et = s_ref[pl.ds(et, 1), :]
    tb = tb_ref[0]
    iota = jax.lax.broadcasted_iota(jnp.int32, (1, N), 1)
    tbu = jnp.sum(jnp.where(iota == ui, tb, 0.0))
    tbv = jnp.sum(jnp.where(iota == vi, tb, 0.0))
    dec_u = alv * jnp.exp(-wtv * ((ti - jnp.maximum(tbu, tb)) / TD_MAX))
    dec_v = alv * jnp.exp(-wtv * ((ti - jnp.maximum(tbv, tb)) / TD_MAX))
    g_u = a_u + s_et + bv + dec_u
    g_v = a_v + s_et + bv + dec_v
    inv_psi = 1.0 / (psv + 1e-7)
    lam_ref[0, :, 0:N] = psv * jnp.log1p(
        jnp.exp(jnp.clip(g_u * inv_psi, -75.0, 75.0)))
    lam_ref[0, :, N:2 * N] = psv * jnp.log1p(
        jnp.exp(jnp.clip(g_v * inv_psi, -75.0, 75.0)))

    hu = h2[0:1, :]
    hv = h2[1:2, :]
    z_ref[pl.ds(ui, 1), :] = hu
    z_ref[pl.ds(vi, 1), :] = hv
    snew = jnp.dot(h2, wb_ref[...].T,
                   preferred_element_type=jnp.float32)
    s0 = s_ref[0:1, :]
    s1 = s_ref[1:2, :]
    s0 = jnp.where(iota == ui, snew[0, 0], s0)
    s1 = jnp.where(iota == ui, snew[0, 1], s1)
    s0 = jnp.where(iota == vi, snew[1, 0], s0)
    s1 = jnp.where(iota == vi, snew[1, 1], s1)
    s_ref[0:1, :] = s0
    s_ref[1:2, :] = s1


@jax.jit
def kernel(u, v, time_diff, event_types, t_bar, t, z0, omega_w, omega_b,
           w_t, alpha, psi, W_struct_w, W_struct_b, W_rec_w, W_rec_b,
           W_t_w, W_t_b):
    u = u.astype(jnp.int32)
    v = v.astype(jnp.int32)
    et = event_types.astype(jnp.int32)
    td8 = time_diff.reshape(B, 8)
    tb3 = t_bar.reshape(B, 1, N)
    wa = omega_w[:, 0, :H]
    wb = omega_w[:, 0, H:]
    ob = omega_b[:, 0]
    bias = (W_struct_b + W_rec_b + W_t_b).reshape(1, H)
    sd4 = jnp.array([50.0, 7.0, 15.0, 15.0], dtype=jnp.float32)
    WtT_n = W_t_w.T / sd4[:, None]

    smem = pl.BlockSpec(memory_space=pltpu.SMEM)
    full = pl.BlockSpec(memory_space=pltpu.VMEM)

    lam3, z_final = pl.pallas_call(
        _scan_kernel,
        grid=(B,),
        in_specs=[
            smem,
            smem,
            smem,
            smem,
            smem,
            smem,
            smem,
            smem,
            full,
            pl.BlockSpec((1, 1, N), lambda i: (i, 0, 0)),
            full,
            full,
            full,
            full,
            full,
            full,
            full,
        ],
        out_specs=[
            pl.BlockSpec((1, 1, 2 * N), lambda i: (i, 0, 0)),
            pl.BlockSpec((N, H), lambda i: (0, 0)),
        ],
        out_shape=[
            jax.ShapeDtypeStruct((B, 1, 2 * N), jnp.float32),
            jax.ShapeDtypeStruct((N, H), jnp.float32),
        ],
        scratch_shapes=[pltpu.VMEM((2, N), jnp.float32)],
        compiler_params=pltpu.CompilerParams(
            dimension_semantics=("arbitrary",)),
    )(u, v, et, t, w_t, alpha, psi, ob, td8, tb3, z0, wa, wb,
      W_struct_w.T, W_rec_w.T, WtT_n, bias)

    return lam3.reshape(B, 2 * N), z_final
